# baseline (speedup 1.0000x reference)
import functools

import jax
import jax.numpy as jnp
from jax import lax
from jax.experimental import pallas as pl
from jax.experimental.pallas import tpu as pltpu
from jax.experimental.pallas import tpu_sc as plsc
from jax.experimental.compute_on import compute_on

from jax._src.interpreters import mlir as _jax_mlir
from jax._src.pallas import mpmd as _jax_mpmd

_jax_mlir._uncacheable_primitives.add(_jax_mpmd.mpmd_map_p)

N = 10000
E = 320000
IN = 128
HID = 64
LAYERS = 10

_NC = 2
_NS = 16
_LANES = 16
NW = _NC * _NS
NPT = 320
NPAD = NW * NPT
C = 256
CS = 128
EP = E + C

_NBLK = 1024
_NG = NPAD // _NBLK


def _full(shape):
    return pl.BlockSpec(shape, lambda *_: (0,) * len(shape))


def _embed_body(x_ref, w0_ref, b0_ref, w1_ref, b1_ref, wmx_ref, bmsg_ref,
                wrx_ref, bro_ref, h_ref, xw_ref, rx_ref):
    i = pl.program_id(0)
    xa = x_ref[...]
    h = jnp.maximum(jnp.dot(xa, w0_ref[...], preferred_element_type=jnp.float32, precision=lax.Precision.HIGHEST) + b0_ref[...], 0.0)
    h = jnp.maximum(jnp.dot(h, w1_ref[...], preferred_element_type=jnp.float32, precision=lax.Precision.HIGHEST) + b1_ref[...], 0.0)
    rx = jnp.dot(xa, wrx_ref[...], preferred_element_type=jnp.float32, precision=lax.Precision.HIGHEST) + bro_ref[...]
    row = i * _NBLK + lax.broadcasted_iota(jnp.int32, (_NBLK, 1), 0)
    keep = row < N
    h_ref[...] = jnp.where(keep, h, 0.0)
    rx_ref[...] = jnp.where(keep, rx, 0.0)
    xw_ref[...] = jnp.dot(xa, wmx_ref[...], preferred_element_type=jnp.float32, precision=lax.Precision.HIGHEST) + bmsg_ref[...]


def _embed(x_pad, W_emb0, b_emb0, W_emb1, b_emb1, W_mx, b_msg, W_rx, b_ro):
    return pl.pallas_call(
        _embed_body,
        grid=(_NG,),
        in_specs=[
            pl.BlockSpec((_NBLK, IN), lambda i: (i, 0)),
            _full((IN, 96)), _full((96,)), _full((96, HID)), _full((HID,)),
            _full((IN, HID)), _full((HID,)), _full((IN, HID)), _full((HID,)),
        ],
        out_specs=[pl.BlockSpec((_NBLK, HID), lambda i: (i, 0))] * 3,
        out_shape=[jax.ShapeDtypeStruct((NPAD, HID), jnp.float32)] * 3,
    )(x_pad, W_emb0, b_emb0, W_emb1, b_emb1, W_mx, b_msg, W_rx, b_ro)


def _eab_body(attr_ref, wme_ref, eab_ref):
    eab_ref[...] = jnp.dot(attr_ref[...], wme_ref[...], preferred_element_type=jnp.float32, precision=lax.Precision.HIGHEST)


def _eab(attr_pad, W_me):
    BLK = 1024
    return pl.pallas_call(
        _eab_body,
        grid=(EP // BLK,),
        in_specs=[pl.BlockSpec((BLK, 4), lambda i: (i, 0)), _full((4, HID))],
        out_specs=pl.BlockSpec((BLK, HID), lambda i: (i, 0)),
        out_shape=jax.ShapeDtypeStruct((EP, HID), jnp.float32),
    )(attr_pad, W_me)


def _t_body(h_ref, xw_ref, wmh_ref, t_ref):
    t = jnp.dot(h_ref[...], wmh_ref[...], preferred_element_type=jnp.float32, precision=lax.Precision.HIGHEST) + xw_ref[...]
    t_ref[...] = jnp.concatenate([t, jnp.zeros((_NBLK, 128 - HID), jnp.float32)], axis=1)


def _t_step(h, xw2, W_mh):
    return pl.pallas_call(
        _t_body,
        grid=(_NG,),
        in_specs=[
            pl.BlockSpec((_NBLK, HID), lambda i: (i, 0)),
            pl.BlockSpec((_NBLK, HID), lambda i: (i, 0)),
            _full((HID, HID)),
        ],
        out_specs=pl.BlockSpec((_NBLK, 128), lambda i: (i, 0)),
        out_shape=jax.ShapeDtypeStruct((NPAD, 128), jnp.float32),
    )(h, xw2, W_mh)


def _sc_layer_body(t_hbm, h_hbm, src_hbm, dst_hbm, eab_hbm, bnd_hbm,
                   hn_hbm, agg, sidx, didx, trows, eav, bv, sem):
    wid = lax.axis_index("s") * _NC + lax.axis_index("c")
    n0 = wid * NPT
    pltpu.sync_copy(bnd_hbm, bv)
    pltpu.sync_copy(h_hbm.at[pl.ds(n0, NPT)], agg.at[pl.ds(0, NPT)])
    bvec = bv[pl.ds(wid, _LANES)]
    b0 = bvec[0]
    b1 = bvec[1]
    start = (b0 // C) * C
    nch = (b1 - start + (C - 1)) // C

    def chunk(k, carry):
        base = start + k * C
        pltpu.sync_copy(src_hbm.at[pl.ds(base, C)], sidx)
        pltpu.sync_copy(dst_hbm.at[pl.ds(base, C)], didx)
        pltpu.sync_copy(eab_hbm.at[pl.ds(base, C)], eav)
        gathers = [
            pltpu.async_copy(t_hbm.at[sidx.at[pl.ds(q * CS, CS)]], trows.at[pl.ds(q * CS, CS)], sem)
            for q in range(C // CS)
        ]
        for g in gathers:
            g.wait()

        def flush(dp, vv):
            dc = jnp.where((dp >= 0) & (dp < NPT), dp, NPT)
            for j in range(HID // _LANES):
                sl = pl.ds(j * _LANES, _LANES)
                agg[dc, sl] = jnp.maximum(agg[dc, sl], vv[j])

        def group(gidx, carry2):
            e0 = gidx * _LANES
            dv = didx[pl.ds(e0, _LANES)] - n0
            for i in range(_LANES):
                d = dv[i]
                e = e0 + i
                m = tuple(
                    jnp.maximum(trows[e, pl.ds(j * _LANES, _LANES)]
                                + eav[e, pl.ds(j * _LANES, _LANES)], 0.0)
                    for j in range(HID // _LANES)
                )

                same = d == carry2[0]

                @pl.when(jnp.logical_not(same))
                def _(c2=carry2):
                    flush(c2[0], c2[1])

                pen = jnp.broadcast_to(
                    jnp.where(same, jnp.float32(0.0), jnp.float32(-jnp.inf)),
                    (_LANES,))
                vv = tuple(
                    jnp.maximum(a + pen, b) for a, b in zip(carry2[1], m)
                )
                carry2 = (d, vv)
            return carry2

        neg = jnp.full((_LANES,), -jnp.inf, jnp.float32)
        carry = lax.fori_loop(
            0, C // _LANES, group,
            (jnp.int32(-1), (neg, neg, neg, neg)))
        flush(carry[0], carry[1])
        return 0

    lax.fori_loop(0, nch, chunk, 0)
    pltpu.sync_copy(agg.at[pl.ds(0, NPT)], hn_hbm.at[pl.ds(n0, NPT)])


_sc_layer = functools.partial(
    pl.kernel,
    mesh=plsc.VectorSubcoreMesh(core_axis_name="c", subcore_axis_name="s"),
    out_type=jax.ShapeDtypeStruct((NPAD, HID), jnp.float32),
    scratch_types=[
        pltpu.VMEM((NPT + 8, HID), jnp.float32),
        pltpu.VMEM((C,), jnp.int32),
        pltpu.VMEM((C,), jnp.int32),
        pltpu.VMEM((C, 128), jnp.float32),
        pltpu.VMEM((C, HID), jnp.float32),
        pltpu.VMEM((48,), jnp.int32),
        pltpu.SemaphoreType.DMA,
    ],
)(_sc_layer_body)


def _ro_body(h_ref, rx_ref, wrh_ref, wc_ref, bc_ref, rem_ref, cr_ref, remo_ref, g_ref):
    i = pl.program_id(0)
    f = jnp.maximum(
        jnp.dot(h_ref[...], wrh_ref[...], preferred_element_type=jnp.float32, precision=lax.Precision.HIGHEST) + rx_ref[...], 0.0)
    part = jnp.broadcast_to(jnp.max(f, axis=0, keepdims=True), (8, HID))

    @pl.when(i == 0)
    def _():
        g_ref[...] = part

    @pl.when(i > 0)
    def _():
        g_ref[...] = jnp.maximum(g_ref[...], part)

    @pl.when(i == _NG - 1)
    def _():
        z = jnp.dot(g_ref[...], wc_ref[...], preferred_element_type=jnp.float32, precision=lax.Precision.HIGHEST) + bc_ref[...]
        c = 1.0 / (1.0 + jnp.exp(-z[0:1]))
        rem = rem_ref[...]
        cr_ref[...] = rem * c
        remo_ref[...] = rem * (1.0 - c)


def _ro(h, rx, W_rh, W_conf, b_conf, rem):
    cr, remo, _ = pl.pallas_call(
        _ro_body,
        grid=(_NG,),
        in_specs=[
            pl.BlockSpec((_NBLK, HID), lambda i: (i, 0)),
            pl.BlockSpec((_NBLK, HID), lambda i: (i, 0)),
            _full((HID, HID)), _full((HID, 1)), _full((1,)), _full((1, 1)),
        ],
        out_specs=[_full((1, 1)), _full((1, 1)), _full((8, HID))],
        out_shape=[
            jax.ShapeDtypeStruct((1, 1), jnp.float32),
            jax.ShapeDtypeStruct((1, 1), jnp.float32),
            jax.ShapeDtypeStruct((8, HID), jnp.float32),
        ],
    )(h, rx, W_rh, W_conf, b_conf, rem)
    return cr, remo


def _tacc_body(h_ref, xw_ref, acc_ref, cr_ref, wmh_ref, t_ref, acco_ref):
    t = jnp.dot(h_ref[...], wmh_ref[...], preferred_element_type=jnp.float32, precision=lax.Precision.HIGHEST) + xw_ref[...]
    t_ref[...] = jnp.concatenate([t, jnp.zeros((_NBLK, 128 - HID), jnp.float32)], axis=1)
    acco_ref[...] = acc_ref[...] + cr_ref[0, 0] * h_ref[...]


def _tacc(h, xw2, acc, cr, W_mh):
    return pl.pallas_call(
        _tacc_body,
        grid=(_NG,),
        in_specs=[
            pl.BlockSpec((_NBLK, HID), lambda i: (i, 0)),
            pl.BlockSpec((_NBLK, HID), lambda i: (i, 0)),
            pl.BlockSpec((_NBLK, HID), lambda i: (i, 0)),
            _full((1, 1)), _full((HID, HID)),
        ],
        out_specs=[
            pl.BlockSpec((_NBLK, 128), lambda i: (i, 0)),
            pl.BlockSpec((_NBLK, HID), lambda i: (i, 0)),
        ],
        out_shape=[
            jax.ShapeDtypeStruct((NPAD, 128), jnp.float32),
            jax.ShapeDtypeStruct((NPAD, HID), jnp.float32),
        ],
    )(h, xw2, acc, cr, W_mh)


def _final_body(acc_ref, rx_ref, wrh_ref, wh_ref, bh_ref, out_ref, g_ref):
    i = pl.program_id(0)
    f = jnp.maximum(
        jnp.dot(acc_ref[...], wrh_ref[...], preferred_element_type=jnp.float32, precision=lax.Precision.HIGHEST) + rx_ref[...], 0.0)
    part = jnp.max(f, axis=0, keepdims=True)

    @pl.when(i == 0)
    def _():
        g_ref[...] = part

    @pl.when(i > 0)
    def _():
        g_ref[...] = jnp.maximum(g_ref[...], part)

    @pl.when(i == _NG - 1)
    def _():
        out_ref[...] = jnp.dot(g_ref[...], wh_ref[...], preferred_element_type=jnp.float32, precision=lax.Precision.HIGHEST) + bh_ref[...]


def _final(acc, rx, W_rh, W_head, b_head):
    out, _ = pl.pallas_call(
        _final_body,
        grid=(_NG,),
        in_specs=[
            pl.BlockSpec((_NBLK, HID), lambda i: (i, 0)),
            pl.BlockSpec((_NBLK, HID), lambda i: (i, 0)),
            _full((HID, HID)), _full((HID, 1)), _full((1,)),
        ],
        out_specs=[_full((1, 1)), _full((1, HID))],
        out_shape=[
            jax.ShapeDtypeStruct((1, 1), jnp.float32),
            jax.ShapeDtypeStruct((1, HID), jnp.float32),
        ],
    )(acc, rx, W_rh, W_head, b_head)
    return out


def kernel(x, edge_index, edge_attr, W_emb0, b_emb0, W_emb1, b_emb1,
           W_msg, b_msg, W_ro, b_ro, W_conf, b_conf, W_head, b_head):
    src = edge_index[0].astype(jnp.int32)
    dst = edge_index[1].astype(jnp.int32)
    W_mh, W_mx, W_me = W_msg[:HID], W_msg[HID:HID + IN], W_msg[HID + IN:]
    W_rh, W_rx = W_ro[:HID], W_ro[HID:]

    dst_s, src_s, a0, a1, a2, a3 = lax.sort(
        (dst, src, edge_attr[:, 0], edge_attr[:, 1], edge_attr[:, 2], edge_attr[:, 3]),
        num_keys=1)
    attr_s = jnp.stack([a0, a1, a2, a3], axis=1)
    src_p = jnp.concatenate([src_s, jnp.zeros((C,), jnp.int32)])
    dst_p = jnp.concatenate([dst_s, jnp.full((C,), NPAD, jnp.int32)])
    attr_p = jnp.concatenate([attr_s, jnp.zeros((C, 4), jnp.float32)])
    bnd = jnp.searchsorted(dst_s, jnp.arange(NW, dtype=jnp.int32) * NPT).astype(jnp.int32)
    bnd = jnp.concatenate([bnd, jnp.full((48 - NW,), E, jnp.int32)])

    x_pad = jnp.pad(x, ((0, NPAD - N), (0, 0)))
    h, xw2, rx = _embed(x_pad, W_emb0, b_emb0, W_emb1, b_emb1, W_mx, b_msg, W_rx, b_ro)
    eab = _eab(attr_p, W_me)

    acc = jnp.zeros((NPAD, HID), jnp.float32)
    rem = jnp.ones((1, 1), jnp.float32)
    t = _t_step(h, xw2, W_mh)
    for _ in range(LAYERS):
        with compute_on('tpu_sparsecore'):
            h = _sc_layer(t, h, src_p, dst_p, eab, bnd)
        cr, rem = _ro(h, rx, W_rh, W_conf, b_conf, rem)
        t, acc = _tacc(h, xw2, acc, cr, W_mh)

    out = _final(acc, rx, W_rh, W_head, b_head)
    return out[0]


# speedup vs baseline: 1.1717x; 1.1717x over previous
import functools

import jax
import jax.numpy as jnp
from jax import lax
from jax.experimental import pallas as pl
from jax.experimental.pallas import tpu as pltpu
from jax.experimental.pallas import tpu_sc as plsc
from jax.experimental.compute_on import compute_on

from jax._src.interpreters import mlir as _jax_mlir
from jax._src.pallas import mpmd as _jax_mpmd

_jax_mlir._uncacheable_primitives.add(_jax_mpmd.mpmd_map_p)

N = 10000
E = 320000
IN = 128
HID = 64
LAYERS = 10

_NC = 2
_NS = 16
_LANES = 16
NW = _NC * _NS
NPT = 320
NPAD = NW * NPT
C = 256
CS = 128
EP = E + C

_NBLK = 1024
_NG = NPAD // _NBLK


def _full(shape):
    return pl.BlockSpec(shape, lambda *_: (0,) * len(shape))


def _embed_body(x_ref, w0_ref, b0_ref, w1_ref, b1_ref, wmx_ref, bmsg_ref,
                wrx_ref, bro_ref, h_ref, xw_ref, rx_ref):
    i = pl.program_id(0)
    xa = x_ref[...]
    h = jnp.maximum(jnp.dot(xa, w0_ref[...], preferred_element_type=jnp.float32, precision=lax.Precision.HIGHEST) + b0_ref[...], 0.0)
    h = jnp.maximum(jnp.dot(h, w1_ref[...], preferred_element_type=jnp.float32, precision=lax.Precision.HIGHEST) + b1_ref[...], 0.0)
    rx = jnp.dot(xa, wrx_ref[...], preferred_element_type=jnp.float32, precision=lax.Precision.HIGHEST) + bro_ref[...]
    row = i * _NBLK + lax.broadcasted_iota(jnp.int32, (_NBLK, 1), 0)
    keep = row < N
    h_ref[...] = jnp.where(keep, h, 0.0)
    rx_ref[...] = jnp.where(keep, rx, 0.0)
    xw_ref[...] = jnp.dot(xa, wmx_ref[...], preferred_element_type=jnp.float32, precision=lax.Precision.HIGHEST) + bmsg_ref[...]


def _embed(x_pad, W_emb0, b_emb0, W_emb1, b_emb1, W_mx, b_msg, W_rx, b_ro):
    return pl.pallas_call(
        _embed_body,
        grid=(_NG,),
        in_specs=[
            pl.BlockSpec((_NBLK, IN), lambda i: (i, 0)),
            _full((IN, 96)), _full((96,)), _full((96, HID)), _full((HID,)),
            _full((IN, HID)), _full((HID,)), _full((IN, HID)), _full((HID,)),
        ],
        out_specs=[pl.BlockSpec((_NBLK, HID), lambda i: (i, 0))] * 3,
        out_shape=[jax.ShapeDtypeStruct((NPAD, HID), jnp.float32)] * 3,
    )(x_pad, W_emb0, b_emb0, W_emb1, b_emb1, W_mx, b_msg, W_rx, b_ro)


def _eab_body(attr_ref, wme_ref, eab_ref):
    eab_ref[...] = jnp.dot(attr_ref[...], wme_ref[...], preferred_element_type=jnp.float32, precision=lax.Precision.HIGHEST)


def _eab(attr_pad, W_me):
    BLK = 1024
    return pl.pallas_call(
        _eab_body,
        grid=(EP // BLK,),
        in_specs=[pl.BlockSpec((BLK, 4), lambda i: (i, 0)), _full((4, HID))],
        out_specs=pl.BlockSpec((BLK, HID), lambda i: (i, 0)),
        out_shape=jax.ShapeDtypeStruct((EP, HID), jnp.float32),
    )(attr_pad, W_me)


def _t_body(h_ref, xw_ref, wmh_ref, t_ref):
    t = jnp.dot(h_ref[...], wmh_ref[...], preferred_element_type=jnp.float32, precision=lax.Precision.HIGHEST) + xw_ref[...]
    t_ref[...] = jnp.concatenate([t, jnp.zeros((_NBLK, 128 - HID), jnp.float32)], axis=1)


def _t_step(h, xw2, W_mh):
    return pl.pallas_call(
        _t_body,
        grid=(_NG,),
        in_specs=[
            pl.BlockSpec((_NBLK, HID), lambda i: (i, 0)),
            pl.BlockSpec((_NBLK, HID), lambda i: (i, 0)),
            _full((HID, HID)),
        ],
        out_specs=pl.BlockSpec((_NBLK, 128), lambda i: (i, 0)),
        out_shape=jax.ShapeDtypeStruct((NPAD, 128), jnp.float32),
    )(h, xw2, W_mh)


def _sc_layer_body(t_hbm, h_hbm, src_hbm, dst_hbm, eab_hbm, bnd_hbm,
                   hn_hbm, agg, sidx, didx, trows, eav, bv, sem, sem2):
    wid = lax.axis_index("s") * _NC + lax.axis_index("c")
    n0 = wid * NPT
    pltpu.sync_copy(bnd_hbm, bv)
    pltpu.sync_copy(h_hbm.at[pl.ds(n0, NPT)], agg.at[pl.ds(0, NPT)])
    bvec = bv[pl.ds(wid, _LANES)]
    b0 = bvec[0]
    b1 = bvec[1]
    start = (b0 // C) * C
    nch = (b1 - start + (C - 1)) // C

    def chunk(k, carry):
        base = start + k * C
        c_sidx = pltpu.async_copy(src_hbm.at[pl.ds(base, C)], sidx, sem)
        c_didx = pltpu.async_copy(dst_hbm.at[pl.ds(base, C)], didx, sem2)
        c_eav = pltpu.async_copy(eab_hbm.at[pl.ds(base, C)], eav, sem2)
        c_sidx.wait()
        gathers = [
            pltpu.async_copy(t_hbm.at[sidx.at[pl.ds(q * CS, CS)]], trows.at[pl.ds(q * CS, CS)], sem2)
            for q in range(C // CS)
        ]
        c_didx.wait()
        c_eav.wait()
        for g in gathers:
            g.wait()

        def flush(dp, vv):
            dc = jnp.where((dp >= 0) & (dp < NPT), dp, NPT)
            for j in range(HID // _LANES):
                sl = pl.ds(j * _LANES, _LANES)
                agg[dc, sl] = jnp.maximum(agg[dc, sl], vv[j])

        def group(gidx, carry2):
            e0 = gidx * _LANES
            dv = didx[pl.ds(e0, _LANES)] - n0
            for i in range(_LANES):
                d = dv[i]
                e = e0 + i
                m = tuple(
                    jnp.maximum(trows[e, pl.ds(j * _LANES, _LANES)]
                                + eav[e, pl.ds(j * _LANES, _LANES)], 0.0)
                    for j in range(HID // _LANES)
                )

                same = d == carry2[0]

                @pl.when(jnp.logical_not(same))
                def _(c2=carry2):
                    flush(c2[0], c2[1])

                pen = jnp.broadcast_to(
                    jnp.where(same, jnp.float32(0.0), jnp.float32(-jnp.inf)),
                    (_LANES,))
                vv = tuple(
                    jnp.maximum(a + pen, b) for a, b in zip(carry2[1], m)
                )
                carry2 = (d, vv)
            return carry2

        neg = jnp.full((_LANES,), -jnp.inf, jnp.float32)
        carry = lax.fori_loop(
            0, C // _LANES, group,
            (jnp.int32(-1), (neg, neg, neg, neg)))
        flush(carry[0], carry[1])
        return 0

    lax.fori_loop(0, nch, chunk, 0)
    pltpu.sync_copy(agg.at[pl.ds(0, NPT)], hn_hbm.at[pl.ds(n0, NPT)])


_sc_layer = functools.partial(
    pl.kernel,
    mesh=plsc.VectorSubcoreMesh(core_axis_name="c", subcore_axis_name="s"),
    out_type=jax.ShapeDtypeStruct((NPAD, HID), jnp.float32),
    scratch_types=[
        pltpu.VMEM((NPT + 8, HID), jnp.float32),
        pltpu.VMEM((C,), jnp.int32),
        pltpu.VMEM((C,), jnp.int32),
        pltpu.VMEM((C, 128), jnp.float32),
        pltpu.VMEM((C, HID), jnp.float32),
        pltpu.VMEM((48,), jnp.int32),
        pltpu.SemaphoreType.DMA,
        pltpu.SemaphoreType.DMA,
    ],
)(_sc_layer_body)


def _ro_body(h_ref, rx_ref, wrh_ref, wc_ref, bc_ref, rem_ref, cr_ref, remo_ref, g_ref):
    i = pl.program_id(0)
    f = jnp.maximum(
        jnp.dot(h_ref[...], wrh_ref[...], preferred_element_type=jnp.float32, precision=lax.Precision.HIGHEST) + rx_ref[...], 0.0)
    part = jnp.broadcast_to(jnp.max(f, axis=0, keepdims=True), (8, HID))

    @pl.when(i == 0)
    def _():
        g_ref[...] = part

    @pl.when(i > 0)
    def _():
        g_ref[...] = jnp.maximum(g_ref[...], part)

    @pl.when(i == _NG - 1)
    def _():
        z = jnp.dot(g_ref[...], wc_ref[...], preferred_element_type=jnp.float32, precision=lax.Precision.HIGHEST) + bc_ref[...]
        c = 1.0 / (1.0 + jnp.exp(-z[0:1]))
        rem = rem_ref[...]
        cr_ref[...] = rem * c
        remo_ref[...] = rem * (1.0 - c)


def _ro(h, rx, W_rh, W_conf, b_conf, rem):
    cr, remo, _ = pl.pallas_call(
        _ro_body,
        grid=(_NG,),
        in_specs=[
            pl.BlockSpec((_NBLK, HID), lambda i: (i, 0)),
            pl.BlockSpec((_NBLK, HID), lambda i: (i, 0)),
            _full((HID, HID)), _full((HID, 1)), _full((1,)), _full((1, 1)),
        ],
        out_specs=[_full((1, 1)), _full((1, 1)), _full((8, HID))],
        out_shape=[
            jax.ShapeDtypeStruct((1, 1), jnp.float32),
            jax.ShapeDtypeStruct((1, 1), jnp.float32),
            jax.ShapeDtypeStruct((8, HID), jnp.float32),
        ],
    )(h, rx, W_rh, W_conf, b_conf, rem)
    return cr, remo


def _tacc_body(h_ref, xw_ref, acc_ref, cr_ref, wmh_ref, t_ref, acco_ref):
    t = jnp.dot(h_ref[...], wmh_ref[...], preferred_element_type=jnp.float32, precision=lax.Precision.HIGHEST) + xw_ref[...]
    t_ref[...] = jnp.concatenate([t, jnp.zeros((_NBLK, 128 - HID), jnp.float32)], axis=1)
    acco_ref[...] = acc_ref[...] + cr_ref[0, 0] * h_ref[...]


def _tacc(h, xw2, acc, cr, W_mh):
    return pl.pallas_call(
        _tacc_body,
        grid=(_NG,),
        in_specs=[
            pl.BlockSpec((_NBLK, HID), lambda i: (i, 0)),
            pl.BlockSpec((_NBLK, HID), lambda i: (i, 0)),
            pl.BlockSpec((_NBLK, HID), lambda i: (i, 0)),
            _full((1, 1)), _full((HID, HID)),
        ],
        out_specs=[
            pl.BlockSpec((_NBLK, 128), lambda i: (i, 0)),
            pl.BlockSpec((_NBLK, HID), lambda i: (i, 0)),
        ],
        out_shape=[
            jax.ShapeDtypeStruct((NPAD, 128), jnp.float32),
            jax.ShapeDtypeStruct((NPAD, HID), jnp.float32),
        ],
    )(h, xw2, acc, cr, W_mh)


def _final_body(acc_ref, rx_ref, wrh_ref, wh_ref, bh_ref, out_ref, g_ref):
    i = pl.program_id(0)
    f = jnp.maximum(
        jnp.dot(acc_ref[...], wrh_ref[...], preferred_element_type=jnp.float32, precision=lax.Precision.HIGHEST) + rx_ref[...], 0.0)
    part = jnp.max(f, axis=0, keepdims=True)

    @pl.when(i == 0)
    def _():
        g_ref[...] = part

    @pl.when(i > 0)
    def _():
        g_ref[...] = jnp.maximum(g_ref[...], part)

    @pl.when(i == _NG - 1)
    def _():
        out_ref[...] = jnp.dot(g_ref[...], wh_ref[...], preferred_element_type=jnp.float32, precision=lax.Precision.HIGHEST) + bh_ref[...]


def _final(acc, rx, W_rh, W_head, b_head):
    out, _ = pl.pallas_call(
        _final_body,
        grid=(_NG,),
        in_specs=[
            pl.BlockSpec((_NBLK, HID), lambda i: (i, 0)),
            pl.BlockSpec((_NBLK, HID), lambda i: (i, 0)),
            _full((HID, HID)), _full((HID, 1)), _full((1,)),
        ],
        out_specs=[_full((1, 1)), _full((1, HID))],
        out_shape=[
            jax.ShapeDtypeStruct((1, 1), jnp.float32),
            jax.ShapeDtypeStruct((1, HID), jnp.float32),
        ],
    )(acc, rx, W_rh, W_head, b_head)
    return out


def kernel(x, edge_index, edge_attr, W_emb0, b_emb0, W_emb1, b_emb1,
           W_msg, b_msg, W_ro, b_ro, W_conf, b_conf, W_head, b_head):
    src = edge_index[0].astype(jnp.int32)
    dst = edge_index[1].astype(jnp.int32)
    W_mh, W_mx, W_me = W_msg[:HID], W_msg[HID:HID + IN], W_msg[HID + IN:]
    W_rh, W_rx = W_ro[:HID], W_ro[HID:]

    dst_s, src_s, a0, a1, a2, a3 = lax.sort(
        (dst, src, edge_attr[:, 0], edge_attr[:, 1], edge_attr[:, 2], edge_attr[:, 3]),
        num_keys=1)
    attr_s = jnp.stack([a0, a1, a2, a3], axis=1)
    src_p = jnp.concatenate([src_s, jnp.zeros((C,), jnp.int32)])
    dst_p = jnp.concatenate([dst_s, jnp.full((C,), NPAD, jnp.int32)])
    attr_p = jnp.concatenate([attr_s, jnp.zeros((C, 4), jnp.float32)])
    bnd = jnp.searchsorted(dst_s, jnp.arange(NW, dtype=jnp.int32) * NPT).astype(jnp.int32)
    bnd = jnp.concatenate([bnd, jnp.full((48 - NW,), E, jnp.int32)])

    x_pad = jnp.pad(x, ((0, NPAD - N), (0, 0)))
    h, xw2, rx = _embed(x_pad, W_emb0, b_emb0, W_emb1, b_emb1, W_mx, b_msg, W_rx, b_ro)
    eab = _eab(attr_p, W_me)

    acc = jnp.zeros((NPAD, HID), jnp.float32)
    rem = jnp.ones((1, 1), jnp.float32)
    t = _t_step(h, xw2, W_mh)
    for _ in range(LAYERS):
        with compute_on('tpu_sparsecore'):
            h = _sc_layer(t, h, src_p, dst_p, eab, bnd)
        cr, rem = _ro(h, rx, W_rh, W_conf, b_conf, rem)
        t, acc = _tacc(h, xw2, acc, cr, W_mh)

    out = _final(acc, rx, W_rh, W_head, b_head)
    return out[0]


# speedup vs baseline: 1.2402x; 1.0585x over previous
import functools

import jax
import jax.numpy as jnp
from jax import lax
from jax.experimental import pallas as pl
from jax.experimental.pallas import tpu as pltpu
from jax.experimental.pallas import tpu_sc as plsc
from jax.experimental.compute_on import compute_on

from jax._src.interpreters import mlir as _jax_mlir
from jax._src.pallas import mpmd as _jax_mpmd

_jax_mlir._uncacheable_primitives.add(_jax_mpmd.mpmd_map_p)

N = 10000
E = 320000
IN = 128
HID = 64
LAYERS = 10

_NC = 2
_NS = 16
_LANES = 16
NW = _NC * _NS
NPT = 320
NPAD = NW * NPT
C = 128
CS = 128
EP = E + 8 * C

_NBLK = 1024
_NG = NPAD // _NBLK


def _full(shape):
    return pl.BlockSpec(shape, lambda *_: (0,) * len(shape))


def _embed_body(x_ref, w0_ref, b0_ref, w1_ref, b1_ref, wmx_ref, bmsg_ref,
                wrx_ref, bro_ref, h_ref, xw_ref, rx_ref):
    i = pl.program_id(0)
    xa = x_ref[...]
    h = jnp.maximum(jnp.dot(xa, w0_ref[...], preferred_element_type=jnp.float32, precision=lax.Precision.HIGHEST) + b0_ref[...], 0.0)
    h = jnp.maximum(jnp.dot(h, w1_ref[...], preferred_element_type=jnp.float32, precision=lax.Precision.HIGHEST) + b1_ref[...], 0.0)
    rx = jnp.dot(xa, wrx_ref[...], preferred_element_type=jnp.float32, precision=lax.Precision.HIGHEST) + bro_ref[...]
    row = i * _NBLK + lax.broadcasted_iota(jnp.int32, (_NBLK, 1), 0)
    keep = row < N
    h_ref[...] = jnp.where(keep, h, 0.0)
    rx_ref[...] = jnp.where(keep, rx, 0.0)
    xw_ref[...] = jnp.dot(xa, wmx_ref[...], preferred_element_type=jnp.float32, precision=lax.Precision.HIGHEST) + bmsg_ref[...]


def _embed(x_pad, W_emb0, b_emb0, W_emb1, b_emb1, W_mx, b_msg, W_rx, b_ro):
    return pl.pallas_call(
        _embed_body,
        grid=(_NG,),
        in_specs=[
            pl.BlockSpec((_NBLK, IN), lambda i: (i, 0)),
            _full((IN, 96)), _full((96,)), _full((96, HID)), _full((HID,)),
            _full((IN, HID)), _full((HID,)), _full((IN, HID)), _full((HID,)),
        ],
        out_specs=[pl.BlockSpec((_NBLK, HID), lambda i: (i, 0))] * 3,
        out_shape=[jax.ShapeDtypeStruct((NPAD, HID), jnp.float32)] * 3,
    )(x_pad, W_emb0, b_emb0, W_emb1, b_emb1, W_mx, b_msg, W_rx, b_ro)


def _eab_body(attr_ref, wme_ref, eab_ref):
    eab_ref[...] = jnp.dot(attr_ref[...], wme_ref[...], preferred_element_type=jnp.float32, precision=lax.Precision.HIGHEST)


def _eab(attr_pad, W_me):
    BLK = 512
    return pl.pallas_call(
        _eab_body,
        grid=(EP // BLK,),
        in_specs=[pl.BlockSpec((BLK, 4), lambda i: (i, 0)), _full((4, HID))],
        out_specs=pl.BlockSpec((BLK, HID), lambda i: (i, 0)),
        out_shape=jax.ShapeDtypeStruct((EP, HID), jnp.float32),
    )(attr_pad, W_me)


def _t_body(h_ref, xw_ref, wmh_ref, t_ref):
    t = jnp.dot(h_ref[...], wmh_ref[...], preferred_element_type=jnp.float32, precision=lax.Precision.HIGHEST) + xw_ref[...]
    t_ref[...] = jnp.concatenate([t, jnp.zeros((_NBLK, 128 - HID), jnp.float32)], axis=1)


def _t_step(h, xw2, W_mh):
    return pl.pallas_call(
        _t_body,
        grid=(_NG,),
        in_specs=[
            pl.BlockSpec((_NBLK, HID), lambda i: (i, 0)),
            pl.BlockSpec((_NBLK, HID), lambda i: (i, 0)),
            _full((HID, HID)),
        ],
        out_specs=pl.BlockSpec((_NBLK, 128), lambda i: (i, 0)),
        out_shape=jax.ShapeDtypeStruct((NPAD, 128), jnp.float32),
    )(h, xw2, W_mh)


def _sc_layer_body(t_hbm, h_hbm, src_hbm, dst_hbm, eab_hbm, bnd_hbm, hn_hbm,
                   agg, sidx0, sidx1, didx0, didx1, eav0, eav1, trows0, trows1,
                   bv, sl0, sl1, sg0, sg1):
    wid = lax.axis_index("s") * _NC + lax.axis_index("c")
    n0 = wid * NPT
    pltpu.sync_copy(bnd_hbm, bv)
    pltpu.sync_copy(h_hbm.at[pl.ds(n0, NPT)], agg.at[pl.ds(0, NPT)])
    bvec = bv[pl.ds(wid, _LANES)]
    b0 = bvec[0]
    b1 = bvec[1]
    start = (b0 // C) * C
    nch = (b1 - start + (C - 1)) // C

    slots = ((sidx0, didx0, eav0, trows0, sl0, sg0),
             (sidx1, didx1, eav1, trows1, sl1, sg1))

    def issue_lin(k, p):
        base = start + k * C
        pltpu.async_copy(src_hbm.at[pl.ds(base, C)], slots[p][0], slots[p][4])
        pltpu.async_copy(dst_hbm.at[pl.ds(base, C)], slots[p][1], slots[p][4])
        pltpu.async_copy(eab_hbm.at[pl.ds(base, C)], slots[p][2], slots[p][4])

    def wait_lin(p):
        pltpu.make_async_copy(src_hbm.at[pl.ds(0, C)], slots[p][0], slots[p][4]).wait()
        pltpu.make_async_copy(dst_hbm.at[pl.ds(0, C)], slots[p][1], slots[p][4]).wait()
        pltpu.make_async_copy(eab_hbm.at[pl.ds(0, C)], slots[p][2], slots[p][4]).wait()

    def issue_gat(p):
        for q in range(C // CS):
            pltpu.async_copy(
                t_hbm.at[slots[p][0].at[pl.ds(q * CS, CS)]],
                slots[p][3].at[pl.ds(q * CS, CS)], slots[p][5])

    def wait_gat(p):
        for q in range(C // CS):
            pltpu.make_async_copy(
                t_hbm.at[slots[p][0].at[pl.ds(q * CS, CS)]],
                slots[p][3].at[pl.ds(q * CS, CS)], slots[p][5]).wait()

    def flush(dp, vv):
        dc = jnp.where((dp >= 0) & (dp < NPT), dp, NPT)
        for j in range(HID // _LANES):
            sl = pl.ds(j * _LANES, _LANES)
            agg[dc, sl] = jnp.maximum(agg[dc, sl], vv[j])

    def compute(p):
        didx, eav, trows = slots[p][1], slots[p][2], slots[p][3]

        def group(gidx, carry2):
            e0 = gidx * _LANES
            dv = didx[pl.ds(e0, _LANES)] - n0
            for i in range(_LANES):
                d = dv[i]
                e = e0 + i
                m = tuple(
                    jnp.maximum(trows[e, pl.ds(j * _LANES, _LANES)]
                                + eav[e, pl.ds(j * _LANES, _LANES)], 0.0)
                    for j in range(HID // _LANES)
                )

                same = d == carry2[0]

                @pl.when(jnp.logical_not(same))
                def _(c2=carry2):
                    flush(c2[0], c2[1])

                pen = jnp.broadcast_to(
                    jnp.where(same, jnp.float32(0.0), jnp.float32(-jnp.inf)),
                    (_LANES,))
                vv = tuple(
                    jnp.maximum(a + pen, b) for a, b in zip(carry2[1], m)
                )
                carry2 = (d, vv)
            return carry2

        neg = jnp.full((_LANES,), -jnp.inf, jnp.float32)
        carry = lax.fori_loop(
            0, C // _LANES, group,
            (jnp.int32(-1), (neg, neg, neg, neg)))
        flush(carry[0], carry[1])

    issue_lin(0, 0)
    wait_lin(0)
    issue_gat(0)
    issue_lin(1, 1)

    def pipe(m, carry):
        k = 2 * m
        wait_lin(1)
        issue_gat(1)
        wait_gat(0)
        compute(0)
        issue_lin(k + 2, 0)
        wait_gat(1)
        compute(1)
        wait_lin(0)
        issue_gat(0)
        issue_lin(k + 3, 1)
        return carry

    lax.fori_loop(0, (nch + 1) // 2, pipe, 0)
    wait_gat(0)
    wait_lin(1)
    pltpu.sync_copy(agg.at[pl.ds(0, NPT)], hn_hbm.at[pl.ds(n0, NPT)])


_sc_layer = functools.partial(
    pl.kernel,
    mesh=plsc.VectorSubcoreMesh(core_axis_name="c", subcore_axis_name="s"),
    out_type=jax.ShapeDtypeStruct((NPAD, HID), jnp.float32),
    scratch_types=[
        pltpu.VMEM((NPT + 8, HID), jnp.float32),
        pltpu.VMEM((C,), jnp.int32),
        pltpu.VMEM((C,), jnp.int32),
        pltpu.VMEM((C,), jnp.int32),
        pltpu.VMEM((C,), jnp.int32),
        pltpu.VMEM((C, HID), jnp.float32),
        pltpu.VMEM((C, HID), jnp.float32),
        pltpu.VMEM((C, 128), jnp.float32),
        pltpu.VMEM((C, 128), jnp.float32),
        pltpu.VMEM((48,), jnp.int32),
        pltpu.SemaphoreType.DMA,
        pltpu.SemaphoreType.DMA,
        pltpu.SemaphoreType.DMA,
        pltpu.SemaphoreType.DMA,
    ],
)(_sc_layer_body)


def _ro_body(h_ref, rx_ref, wrh_ref, wc_ref, bc_ref, rem_ref, cr_ref, remo_ref, g_ref):
    i = pl.program_id(0)
    f = jnp.maximum(
        jnp.dot(h_ref[...], wrh_ref[...], preferred_element_type=jnp.float32, precision=lax.Precision.HIGHEST) + rx_ref[...], 0.0)
    part = jnp.broadcast_to(jnp.max(f, axis=0, keepdims=True), (8, HID))

    @pl.when(i == 0)
    def _():
        g_ref[...] = part

    @pl.when(i > 0)
    def _():
        g_ref[...] = jnp.maximum(g_ref[...], part)

    @pl.when(i == _NG - 1)
    def _():
        z = jnp.dot(g_ref[...], wc_ref[...], preferred_element_type=jnp.float32, precision=lax.Precision.HIGHEST) + bc_ref[...]
        c = 1.0 / (1.0 + jnp.exp(-z[0:1]))
        rem = rem_ref[...]
        cr_ref[...] = rem * c
        remo_ref[...] = rem * (1.0 - c)


def _ro(h, rx, W_rh, W_conf, b_conf, rem):
    cr, remo, _ = pl.pallas_call(
        _ro_body,
        grid=(_NG,),
        in_specs=[
            pl.BlockSpec((_NBLK, HID), lambda i: (i, 0)),
            pl.BlockSpec((_NBLK, HID), lambda i: (i, 0)),
            _full((HID, HID)), _full((HID, 1)), _full((1,)), _full((1, 1)),
        ],
        out_specs=[_full((1, 1)), _full((1, 1)), _full((8, HID))],
        out_shape=[
            jax.ShapeDtypeStruct((1, 1), jnp.float32),
            jax.ShapeDtypeStruct((1, 1), jnp.float32),
            jax.ShapeDtypeStruct((8, HID), jnp.float32),
        ],
    )(h, rx, W_rh, W_conf, b_conf, rem)
    return cr, remo


def _tacc_body(h_ref, xw_ref, acc_ref, cr_ref, wmh_ref, t_ref, acco_ref):
    t = jnp.dot(h_ref[...], wmh_ref[...], preferred_element_type=jnp.float32, precision=lax.Precision.HIGHEST) + xw_ref[...]
    t_ref[...] = jnp.concatenate([t, jnp.zeros((_NBLK, 128 - HID), jnp.float32)], axis=1)
    acco_ref[...] = acc_ref[...] + cr_ref[0, 0] * h_ref[...]


def _tacc(h, xw2, acc, cr, W_mh):
    return pl.pallas_call(
        _tacc_body,
        grid=(_NG,),
        in_specs=[
            pl.BlockSpec((_NBLK, HID), lambda i: (i, 0)),
            pl.BlockSpec((_NBLK, HID), lambda i: (i, 0)),
            pl.BlockSpec((_NBLK, HID), lambda i: (i, 0)),
            _full((1, 1)), _full((HID, HID)),
        ],
        out_specs=[
            pl.BlockSpec((_NBLK, 128), lambda i: (i, 0)),
            pl.BlockSpec((_NBLK, HID), lambda i: (i, 0)),
        ],
        out_shape=[
            jax.ShapeDtypeStruct((NPAD, 128), jnp.float32),
            jax.ShapeDtypeStruct((NPAD, HID), jnp.float32),
        ],
    )(h, xw2, acc, cr, W_mh)


def _final_body(acc_ref, rx_ref, wrh_ref, wh_ref, bh_ref, out_ref, g_ref):
    i = pl.program_id(0)
    f = jnp.maximum(
        jnp.dot(acc_ref[...], wrh_ref[...], preferred_element_type=jnp.float32, precision=lax.Precision.HIGHEST) + rx_ref[...], 0.0)
    part = jnp.max(f, axis=0, keepdims=True)

    @pl.when(i == 0)
    def _():
        g_ref[...] = part

    @pl.when(i > 0)
    def _():
        g_ref[...] = jnp.maximum(g_ref[...], part)

    @pl.when(i == _NG - 1)
    def _():
        out_ref[...] = jnp.dot(g_ref[...], wh_ref[...], preferred_element_type=jnp.float32, precision=lax.Precision.HIGHEST) + bh_ref[...]


def _final(acc, rx, W_rh, W_head, b_head):
    out, _ = pl.pallas_call(
        _final_body,
        grid=(_NG,),
        in_specs=[
            pl.BlockSpec((_NBLK, HID), lambda i: (i, 0)),
            pl.BlockSpec((_NBLK, HID), lambda i: (i, 0)),
            _full((HID, HID)), _full((HID, 1)), _full((1,)),
        ],
        out_specs=[_full((1, 1)), _full((1, HID))],
        out_shape=[
            jax.ShapeDtypeStruct((1, 1), jnp.float32),
            jax.ShapeDtypeStruct((1, HID), jnp.float32),
        ],
    )(acc, rx, W_rh, W_head, b_head)
    return out


def kernel(x, edge_index, edge_attr, W_emb0, b_emb0, W_emb1, b_emb1,
           W_msg, b_msg, W_ro, b_ro, W_conf, b_conf, W_head, b_head):
    src = edge_index[0].astype(jnp.int32)
    dst = edge_index[1].astype(jnp.int32)
    W_mh, W_mx, W_me = W_msg[:HID], W_msg[HID:HID + IN], W_msg[HID + IN:]
    W_rh, W_rx = W_ro[:HID], W_ro[HID:]

    dst_s, src_s, a0, a1, a2, a3 = lax.sort(
        (dst, src, edge_attr[:, 0], edge_attr[:, 1], edge_attr[:, 2], edge_attr[:, 3]),
        num_keys=1)
    attr_s = jnp.stack([a0, a1, a2, a3], axis=1)
    src_p = jnp.concatenate([src_s, jnp.zeros((EP - E,), jnp.int32)])
    dst_p = jnp.concatenate([dst_s, jnp.full((EP - E,), NPAD, jnp.int32)])
    attr_p = jnp.concatenate([attr_s, jnp.zeros((EP - E, 4), jnp.float32)])
    bnd = jnp.searchsorted(dst_s, jnp.arange(NW, dtype=jnp.int32) * NPT).astype(jnp.int32)
    bnd = jnp.concatenate([bnd, jnp.full((48 - NW,), E, jnp.int32)])

    x_pad = jnp.pad(x, ((0, NPAD - N), (0, 0)))
    h, xw2, rx = _embed(x_pad, W_emb0, b_emb0, W_emb1, b_emb1, W_mx, b_msg, W_rx, b_ro)
    eab = _eab(attr_p, W_me)

    acc = jnp.zeros((NPAD, HID), jnp.float32)
    rem = jnp.ones((1, 1), jnp.float32)
    t = _t_step(h, xw2, W_mh)
    for _ in range(LAYERS):
        with compute_on('tpu_sparsecore'):
            h = _sc_layer(t, h, src_p, dst_p, eab, bnd)
        cr, rem = _ro(h, rx, W_rh, W_conf, b_conf, rem)
        t, acc = _tacc(h, xw2, acc, cr, W_mh)

    out = _final(acc, rx, W_rh, W_head, b_head)
    return out[0]


# speedup vs baseline: 1.2663x; 1.0211x over previous
import functools

import jax
import jax.numpy as jnp
from jax import lax
from jax.experimental import pallas as pl
from jax.experimental.pallas import tpu as pltpu
from jax.experimental.pallas import tpu_sc as plsc
from jax.experimental.compute_on import compute_on

from jax._src.interpreters import mlir as _jax_mlir
from jax._src.pallas import mpmd as _jax_mpmd

_jax_mlir._uncacheable_primitives.add(_jax_mpmd.mpmd_map_p)

N = 10000
E = 320000
IN = 128
HID = 64
LAYERS = 10

_NC = 2
_NS = 16
_LANES = 16
NW = _NC * _NS
NPT = 320
NPAD = NW * NPT
C = 128
CS = 128
EP = E + 8 * C

_NBLK = 1024
_NG = NPAD // _NBLK


def _full(shape):
    return pl.BlockSpec(shape, lambda *_: (0,) * len(shape))


def _embed_body(x_ref, w0_ref, b0_ref, w1_ref, b1_ref, wmx_ref, bmsg_ref,
                wrx_ref, bro_ref, h_ref, xw_ref, rx_ref):
    i = pl.program_id(0)
    xa = x_ref[...]
    h = jnp.maximum(jnp.dot(xa, w0_ref[...], preferred_element_type=jnp.float32, precision=lax.Precision.HIGHEST) + b0_ref[...], 0.0)
    h = jnp.maximum(jnp.dot(h, w1_ref[...], preferred_element_type=jnp.float32, precision=lax.Precision.HIGHEST) + b1_ref[...], 0.0)
    rx = jnp.dot(xa, wrx_ref[...], preferred_element_type=jnp.float32, precision=lax.Precision.HIGHEST) + bro_ref[...]
    row = i * _NBLK + lax.broadcasted_iota(jnp.int32, (_NBLK, 1), 0)
    keep = row < N
    h_ref[...] = jnp.where(keep, h, 0.0)
    rx_ref[...] = jnp.where(keep, rx, 0.0)
    xw_ref[...] = jnp.dot(xa, wmx_ref[...], preferred_element_type=jnp.float32, precision=lax.Precision.HIGHEST) + bmsg_ref[...]


def _embed(x_pad, W_emb0, b_emb0, W_emb1, b_emb1, W_mx, b_msg, W_rx, b_ro):
    return pl.pallas_call(
        _embed_body,
        grid=(_NG,),
        in_specs=[
            pl.BlockSpec((_NBLK, IN), lambda i: (i, 0)),
            _full((IN, 96)), _full((96,)), _full((96, HID)), _full((HID,)),
            _full((IN, HID)), _full((HID,)), _full((IN, HID)), _full((HID,)),
        ],
        out_specs=[pl.BlockSpec((_NBLK, HID), lambda i: (i, 0))] * 3,
        out_shape=[jax.ShapeDtypeStruct((NPAD, HID), jnp.float32)] * 3,
    )(x_pad, W_emb0, b_emb0, W_emb1, b_emb1, W_mx, b_msg, W_rx, b_ro)


def _eab_body(attr_ref, wme_ref, eab_ref):
    eab_ref[...] = jnp.dot(attr_ref[...], wme_ref[...], preferred_element_type=jnp.float32, precision=lax.Precision.HIGHEST)


def _eab(attr_pad, W_me):
    BLK = 512
    return pl.pallas_call(
        _eab_body,
        grid=(EP // BLK,),
        in_specs=[pl.BlockSpec((BLK, 4), lambda i: (i, 0)), _full((4, HID))],
        out_specs=pl.BlockSpec((BLK, HID), lambda i: (i, 0)),
        out_shape=jax.ShapeDtypeStruct((EP, HID), jnp.float32),
    )(attr_pad, W_me)


def _t_body(h_ref, xw_ref, wmh_ref, t_ref):
    t = jnp.dot(h_ref[...], wmh_ref[...], preferred_element_type=jnp.float32, precision=lax.Precision.HIGHEST) + xw_ref[...]
    t_ref[...] = jnp.concatenate([t, jnp.zeros((_NBLK, 128 - HID), jnp.float32)], axis=1)


def _t_step(h, xw2, W_mh):
    return pl.pallas_call(
        _t_body,
        grid=(_NG,),
        in_specs=[
            pl.BlockSpec((_NBLK, HID), lambda i: (i, 0)),
            pl.BlockSpec((_NBLK, HID), lambda i: (i, 0)),
            _full((HID, HID)),
        ],
        out_specs=pl.BlockSpec((_NBLK, 128), lambda i: (i, 0)),
        out_shape=jax.ShapeDtypeStruct((NPAD, 128), jnp.float32),
    )(h, xw2, W_mh)


def _sc_layer_body(t_hbm, h_hbm, src_hbm, dst_hbm, eab_hbm, bnd_hbm, hn_hbm,
                   agg, sidx0, sidx1, didx0, didx1, eav0, eav1, trows0, trows1,
                   bv, sl0, sl1, sg0, sg1):
    wid = lax.axis_index("s") * _NC + lax.axis_index("c")
    n0 = wid * NPT
    pltpu.sync_copy(bnd_hbm, bv)
    pltpu.sync_copy(h_hbm.at[pl.ds(n0, NPT)], agg.at[pl.ds(0, NPT)])
    bvec = bv[pl.ds(wid, _LANES)]
    b0 = bvec[0]
    b1 = bvec[1]
    start = (b0 // C) * C
    nch = (b1 - start + (C - 1)) // C

    slots = ((sidx0, didx0, eav0, trows0, sl0, sg0),
             (sidx1, didx1, eav1, trows1, sl1, sg1))

    def issue_lin(k, p):
        base = start + k * C
        pltpu.async_copy(src_hbm.at[pl.ds(base, C)], slots[p][0], slots[p][4])
        pltpu.async_copy(dst_hbm.at[pl.ds(base, C)], slots[p][1], slots[p][4])
        pltpu.async_copy(eab_hbm.at[pl.ds(base, C)], slots[p][2], slots[p][4])

    def wait_lin(p):
        pltpu.make_async_copy(src_hbm.at[pl.ds(0, C)], slots[p][0], slots[p][4]).wait()
        pltpu.make_async_copy(dst_hbm.at[pl.ds(0, C)], slots[p][1], slots[p][4]).wait()
        pltpu.make_async_copy(eab_hbm.at[pl.ds(0, C)], slots[p][2], slots[p][4]).wait()

    def issue_gat(p):
        for q in range(C // CS):
            pltpu.async_copy(
                t_hbm.at[slots[p][0].at[pl.ds(q * CS, CS)]],
                slots[p][3].at[pl.ds(q * CS, CS)], slots[p][5])

    def wait_gat(p):
        for q in range(C // CS):
            pltpu.make_async_copy(
                t_hbm.at[slots[p][0].at[pl.ds(q * CS, CS)]],
                slots[p][3].at[pl.ds(q * CS, CS)], slots[p][5]).wait()

    def flush(dp, vv):
        dc = jnp.where((dp >= 0) & (dp < NPT), dp, NPT)
        for j in range(HID // _LANES):
            sl = pl.ds(j * _LANES, _LANES)
            agg[dc, sl] = jnp.maximum(agg[dc, sl], vv[j])

    def compute(p):
        didx, eav, trows = slots[p][1], slots[p][2], slots[p][3]

        def group(gidx, carry2):
            e0 = gidx * _LANES
            dv = didx[pl.ds(e0, _LANES)] - n0
            for i in range(_LANES):
                d = dv[i]
                e = e0 + i
                m = tuple(
                    jnp.maximum(trows[e, pl.ds(j * _LANES, _LANES)]
                                + eav[e, pl.ds(j * _LANES, _LANES)], 0.0)
                    for j in range(HID // _LANES)
                )

                same = d == carry2[0]

                @pl.when(jnp.logical_not(same))
                def _(c2=carry2):
                    flush(c2[0], c2[1])

                pen = jnp.broadcast_to(
                    jnp.where(same, jnp.float32(0.0), jnp.float32(-jnp.inf)),
                    (_LANES,))
                vv = tuple(
                    jnp.maximum(a + pen, b) for a, b in zip(carry2[1], m)
                )
                carry2 = (d, vv)
            return carry2

        neg = jnp.full((_LANES,), -jnp.inf, jnp.float32)
        carry = lax.fori_loop(
            0, C // _LANES, group,
            (jnp.int32(-1), (neg, neg, neg, neg)))
        flush(carry[0], carry[1])

    issue_lin(0, 0)
    wait_lin(0)
    issue_gat(0)
    issue_lin(1, 1)

    def pipe(m, carry):
        k = 2 * m
        wait_lin(1)
        issue_gat(1)
        wait_gat(0)
        compute(0)
        issue_lin(k + 2, 0)
        wait_gat(1)
        compute(1)
        wait_lin(0)
        issue_gat(0)
        issue_lin(k + 3, 1)
        return carry

    lax.fori_loop(0, (nch + 1) // 2, pipe, 0)
    wait_gat(0)
    wait_lin(1)
    pltpu.sync_copy(agg.at[pl.ds(0, NPT)], hn_hbm.at[pl.ds(n0, NPT)])


_sc_layer = functools.partial(
    pl.kernel,
    mesh=plsc.VectorSubcoreMesh(core_axis_name="c", subcore_axis_name="s"),
    out_type=jax.ShapeDtypeStruct((NPAD, HID), jnp.float32),
    scratch_types=[
        pltpu.VMEM((NPT + 8, HID), jnp.float32),
        pltpu.VMEM((C,), jnp.int32),
        pltpu.VMEM((C,), jnp.int32),
        pltpu.VMEM((C,), jnp.int32),
        pltpu.VMEM((C,), jnp.int32),
        pltpu.VMEM((C, HID), jnp.float32),
        pltpu.VMEM((C, HID), jnp.float32),
        pltpu.VMEM((C, 128), jnp.float32),
        pltpu.VMEM((C, 128), jnp.float32),
        pltpu.VMEM((48,), jnp.int32),
        pltpu.SemaphoreType.DMA,
        pltpu.SemaphoreType.DMA,
        pltpu.SemaphoreType.DMA,
        pltpu.SemaphoreType.DMA,
    ],
)(_sc_layer_body)


def _ro_body(h_ref, rx_ref, wrh_ref, wc_ref, bc_ref, rem_ref, cr_ref, remo_ref, g_ref):
    i = pl.program_id(0)
    f = jnp.maximum(
        jnp.dot(h_ref[...], wrh_ref[...], preferred_element_type=jnp.float32, precision=lax.Precision.HIGHEST) + rx_ref[...], 0.0)
    part = jnp.broadcast_to(jnp.max(f, axis=0, keepdims=True), (8, HID))

    @pl.when(i == 0)
    def _():
        g_ref[...] = part

    @pl.when(i > 0)
    def _():
        g_ref[...] = jnp.maximum(g_ref[...], part)

    @pl.when(i == _NG - 1)
    def _():
        z = jnp.dot(g_ref[...], wc_ref[...], preferred_element_type=jnp.float32, precision=lax.Precision.HIGHEST) + bc_ref[...]
        c = 1.0 / (1.0 + jnp.exp(-z[0:1]))
        rem = rem_ref[...]
        cr_ref[...] = rem * c
        remo_ref[...] = rem * (1.0 - c)


def _ro(h, rx, W_rh, W_conf, b_conf, rem):
    cr, remo, _ = pl.pallas_call(
        _ro_body,
        grid=(_NG,),
        in_specs=[
            pl.BlockSpec((_NBLK, HID), lambda i: (i, 0)),
            pl.BlockSpec((_NBLK, HID), lambda i: (i, 0)),
            _full((HID, HID)), _full((HID, 1)), _full((1,)), _full((1, 1)),
        ],
        out_specs=[_full((1, 1)), _full((1, 1)), _full((8, HID))],
        out_shape=[
            jax.ShapeDtypeStruct((1, 1), jnp.float32),
            jax.ShapeDtypeStruct((1, 1), jnp.float32),
            jax.ShapeDtypeStruct((8, HID), jnp.float32),
        ],
    )(h, rx, W_rh, W_conf, b_conf, rem)
    return cr, remo


def _tacc_body(h_ref, xw_ref, acc_ref, cr_ref, wmh_ref, t_ref, acco_ref):
    t = jnp.dot(h_ref[...], wmh_ref[...], preferred_element_type=jnp.float32, precision=lax.Precision.HIGHEST) + xw_ref[...]
    t_ref[...] = jnp.concatenate([t, jnp.zeros((_NBLK, 128 - HID), jnp.float32)], axis=1)
    acco_ref[...] = acc_ref[...] + cr_ref[0, 0] * h_ref[...]


def _tacc(h, xw2, acc, cr, W_mh):
    return pl.pallas_call(
        _tacc_body,
        grid=(_NG,),
        in_specs=[
            pl.BlockSpec((_NBLK, HID), lambda i: (i, 0)),
            pl.BlockSpec((_NBLK, HID), lambda i: (i, 0)),
            pl.BlockSpec((_NBLK, HID), lambda i: (i, 0)),
            _full((1, 1)), _full((HID, HID)),
        ],
        out_specs=[
            pl.BlockSpec((_NBLK, 128), lambda i: (i, 0)),
            pl.BlockSpec((_NBLK, HID), lambda i: (i, 0)),
        ],
        out_shape=[
            jax.ShapeDtypeStruct((NPAD, 128), jnp.float32),
            jax.ShapeDtypeStruct((NPAD, HID), jnp.float32),
        ],
    )(h, xw2, acc, cr, W_mh)


def _final_body(acc_ref, rx_ref, wrh_ref, wh_ref, bh_ref, out_ref, g_ref):
    i = pl.program_id(0)
    f = jnp.maximum(
        jnp.dot(acc_ref[...], wrh_ref[...], preferred_element_type=jnp.float32, precision=lax.Precision.HIGHEST) + rx_ref[...], 0.0)
    part = jnp.max(f, axis=0, keepdims=True)

    @pl.when(i == 0)
    def _():
        g_ref[...] = part

    @pl.when(i > 0)
    def _():
        g_ref[...] = jnp.maximum(g_ref[...], part)

    @pl.when(i == _NG - 1)
    def _():
        out_ref[...] = jnp.dot(g_ref[...], wh_ref[...], preferred_element_type=jnp.float32, precision=lax.Precision.HIGHEST) + bh_ref[...]


def _final(acc, rx, W_rh, W_head, b_head):
    out, _ = pl.pallas_call(
        _final_body,
        grid=(_NG,),
        in_specs=[
            pl.BlockSpec((_NBLK, HID), lambda i: (i, 0)),
            pl.BlockSpec((_NBLK, HID), lambda i: (i, 0)),
            _full((HID, HID)), _full((HID, 1)), _full((1,)),
        ],
        out_specs=[_full((1, 1)), _full((1, HID))],
        out_shape=[
            jax.ShapeDtypeStruct((1, 1), jnp.float32),
            jax.ShapeDtypeStruct((1, HID), jnp.float32),
        ],
    )(acc, rx, W_rh, W_head, b_head)
    return out


def kernel(x, edge_index, edge_attr, W_emb0, b_emb0, W_emb1, b_emb1,
           W_msg, b_msg, W_ro, b_ro, W_conf, b_conf, W_head, b_head):
    src = edge_index[0].astype(jnp.int32)
    dst = edge_index[1].astype(jnp.int32)
    W_mh, W_mx, W_me = W_msg[:HID], W_msg[HID:HID + IN], W_msg[HID + IN:]
    W_rh, W_rx = W_ro[:HID], W_ro[HID:]

    dst_s, perm = lax.sort((dst, jnp.arange(E, dtype=jnp.int32)), num_keys=1)
    src_s = jnp.take(src, perm)
    attr_s = jnp.take(edge_attr, perm, axis=0)
    src_p = jnp.concatenate([src_s, jnp.zeros((EP - E,), jnp.int32)])
    dst_p = jnp.concatenate([dst_s, jnp.full((EP - E,), NPAD, jnp.int32)])
    attr_p = jnp.concatenate([attr_s, jnp.zeros((EP - E, 4), jnp.float32)])
    bnd = jnp.searchsorted(dst_s, jnp.arange(NW, dtype=jnp.int32) * NPT).astype(jnp.int32)
    bnd = jnp.concatenate([bnd, jnp.full((48 - NW,), E, jnp.int32)])

    x_pad = jnp.pad(x, ((0, NPAD - N), (0, 0)))
    h, xw2, rx = _embed(x_pad, W_emb0, b_emb0, W_emb1, b_emb1, W_mx, b_msg, W_rx, b_ro)
    eab = _eab(attr_p, W_me)

    acc = jnp.zeros((NPAD, HID), jnp.float32)
    rem = jnp.ones((1, 1), jnp.float32)
    t = _t_step(h, xw2, W_mh)
    for _ in range(LAYERS):
        with compute_on('tpu_sparsecore'):
            h = _sc_layer(t, h, src_p, dst_p, eab, bnd)
        cr, rem = _ro(h, rx, W_rh, W_conf, b_conf, rem)
        t, acc = _tacc(h, xw2, acc, cr, W_mh)

    out = _final(acc, rx, W_rh, W_head, b_head)
    return out[0]
